# Initial kernel scaffold; baseline (speedup 1.0000x reference)
import jax
import jax.numpy as jnp
from jax.experimental import pallas as pl


def kernel(data, w1, w2, w3, w4, w5):
    raise NotImplementedError("write your pallas kernel here")


# speedup vs baseline: 2.8762x; 2.8762x over previous
import functools
import jax
import jax.numpy as jnp
from jax.experimental import pallas as pl
from jax.experimental.pallas import tpu as pltpu

_PD = 8
_G = 8
_DS = 8


def _encoder_kernel(S, data_ref, w_ref, prob_ref, regr_ref, a_ref, b_ref):
    SG = S + _G
    SD = S + 2 * _PD
    nslab = S // _DS
    bi = pl.program_id(0)

    @pl.when(bi == 0)
    def _():
        def zero_plane(d, _):
            z = jnp.zeros((SG, SG), jnp.float32)
            a_ref[0, d] = z
            a_ref[1, d] = z
            b_ref[0, d] = z
            b_ref[1, d] = z
            return 0
        jax.lax.fori_loop(0, SD, zero_plane, 0)

    def copy_plane(d, _):
        a_ref[0, _PD + d, 0:S, 0:S] = data_ref[0, d]
        return 0
    jax.lax.fori_loop(0, S, copy_plane, 0)

    layers = [
        (0, 1, 1, a_ref, b_ref),
        (54, 2, 2, b_ref, a_ref),
        (162, 2, 4, a_ref, b_ref),
        (270, 2, 2, b_ref, None),
    ]

    for base, cin, dil, src, dst in layers:
        def slab_body(ds, _, base=base, cin=cin, dil=dil, src=src, dst=dst):
            d0 = ds * _DS

            def kd_body(kd, acc, base=base, cin=cin, dil=dil, src=src):
                sd = _PD + d0 + dil * (kd - 1)
                xs = [src[ci, pl.ds(sd, _DS), :, :] for ci in range(cin)]

                def tap(j, acc2, base=base, cin=cin, dil=dil, kd=kd, xs=xs):
                    kh = j // 3
                    kw = j - 3 * kh
                    ah = jax.lax.rem(SG - dil * (kh - 1), SG)
                    aw = jax.lax.rem(SG - dil * (kw - 1), SG)
                    t = kd * 9 + j
                    a0, a1 = acc2
                    for ci in range(cin):
                        v = pltpu.roll(xs[ci], ah, 1)
                        v = pltpu.roll(v, aw, 2)
                        a0 = a0 + w_ref[base + ci * 27 + t] * v
                        a1 = a1 + w_ref[base + (cin + ci) * 27 + t] * v
                    return (a0, a1)

                return jax.lax.fori_loop(0, 9, tap, acc)

            z = jnp.zeros((_DS, SG, SG), jnp.float32)
            acc0, acc1 = jax.lax.fori_loop(0, 3, kd_body, (z, z))
            acc0 = acc0[:, 0:S, 0:S]
            acc1 = acc1[:, 0:S, 0:S]
            m = (data_ref[0, pl.ds(d0, _DS)] != 0.0).astype(jnp.float32)
            if dst is not None:
                dst[0, pl.ds(_PD + d0, _DS), 0:S, 0:S] = m * jnp.maximum(acc0, 0.0)
                dst[1, pl.ds(_PD + d0, _DS), 0:S, 0:S] = m * jnp.maximum(acc1, 0.0)
            else:
                x0 = m * jnp.maximum(acc0, 0.0)
                x1 = m * jnp.maximum(acc1, 0.0)
                p = w_ref[378] * x0 + w_ref[379] * x1
                q = w_ref[380] * x0 + w_ref[381] * x1
                prob_ref[0, 0, pl.ds(d0, _DS)] = m / (1.0 + jnp.exp(-p))
                regr_ref[0, 0, pl.ds(d0, _DS)] = m / (1.0 + jnp.exp(-q))
            return 0

        jax.lax.fori_loop(0, nslab, slab_body, 0)


def kernel(data, w1, w2, w3, w4, w5):
    B, S = data.shape[0], data.shape[1]
    wflat = jnp.concatenate([
        w1.reshape(-1), w2.reshape(-1), w3.reshape(-1), w4.reshape(-1),
        w5.reshape(-1)])
    SG = S + _G
    SD = S + 2 * _PD
    out_shape = jax.ShapeDtypeStruct((B, 1, S, S, S), jnp.float32)
    prob, regr = pl.pallas_call(
        functools.partial(_encoder_kernel, S),
        grid=(B,),
        in_specs=[
            pl.BlockSpec((1, S, S, S), lambda b: (b, 0, 0, 0)),
            pl.BlockSpec(memory_space=pltpu.SMEM),
        ],
        out_specs=[
            pl.BlockSpec((1, 1, S, S, S), lambda b: (b, 0, 0, 0, 0)),
            pl.BlockSpec((1, 1, S, S, S), lambda b: (b, 0, 0, 0, 0)),
        ],
        out_shape=[out_shape, out_shape],
        scratch_shapes=[
            pltpu.VMEM((2, SD, SG, SG), jnp.float32),
            pltpu.VMEM((2, SD, SG, SG), jnp.float32),
        ],
    )(data, wflat)
    return (prob, regr)


# speedup vs baseline: 4.0614x; 1.4121x over previous
import functools
import jax
import jax.numpy as jnp
from jax.experimental import pallas as pl
from jax.experimental.pallas import tpu as pltpu

_PD = 4
_G = 8
_DS = 8


def _encoder_kernel(S, data_ref, w_ref, prob_ref, regr_ref, a_ref, b_ref):
    SG = S + _G
    SD = S + 2 * _PD
    nslab = S // _DS

    def zero_plane(d, _):
        z = jnp.zeros((SG, SG), jnp.float32)
        a_ref[0, d] = z
        a_ref[1, d] = z
        b_ref[0, d] = z
        b_ref[1, d] = z
        return 0
    jax.lax.fori_loop(0, SD, zero_plane, 0)

    def copy_plane(d, _):
        a_ref[0, _PD + d, 0:S, 0:S] = data_ref[d]
        return 0
    jax.lax.fori_loop(0, S, copy_plane, 0)

    layers = [
        (0, 1, 1, a_ref, b_ref),
        (54, 2, 2, b_ref, a_ref),
        (162, 2, 4, a_ref, b_ref),
        (270, 2, 2, b_ref, None),
    ]

    for base, cin, dil, src, dst in layers:
        def slab_body(ds, _, base=base, cin=cin, dil=dil, src=src, dst=dst):
            d0 = ds * _DS

            def kd_body(kd, acc, base=base, cin=cin, dil=dil, src=src):
                sd = _PD + d0 + dil * (kd - 1)
                a0, a1 = acc
                for ci in range(cin):
                    xd = src[ci, pl.ds(sd, _DS), :, :]
                    for kh in range(3):
                        if kh == 1:
                            xh = xd
                        else:
                            xh = pltpu.roll(xd, (SG - dil * (kh - 1)) % SG, 1)
                        for kw in range(3):
                            if kw == 1:
                                xw = xh
                            else:
                                xw = pltpu.roll(xh, (SG - dil * (kw - 1)) % SG, 2)
                            t = kd * 9 + kh * 3 + kw
                            a0 = a0 + w_ref[base + ci * 27 + t] * xw
                            a1 = a1 + w_ref[base + (cin + ci) * 27 + t] * xw
                return (a0, a1)

            z = jnp.zeros((_DS, SG, SG), jnp.float32)
            acc0, acc1 = jax.lax.fori_loop(0, 3, kd_body, (z, z))
            acc0 = acc0[:, 0:S, 0:S]
            acc1 = acc1[:, 0:S, 0:S]
            m = (data_ref[pl.ds(d0, _DS)] != 0.0).astype(jnp.float32)
            if dst is not None:
                dst[0, pl.ds(_PD + d0, _DS), 0:S, 0:S] = m * jnp.maximum(acc0, 0.0)
                dst[1, pl.ds(_PD + d0, _DS), 0:S, 0:S] = m * jnp.maximum(acc1, 0.0)
            else:
                x0 = m * jnp.maximum(acc0, 0.0)
                x1 = m * jnp.maximum(acc1, 0.0)
                p = w_ref[378] * x0 + w_ref[379] * x1
                q = w_ref[380] * x0 + w_ref[381] * x1
                prob_ref[pl.ds(d0, _DS)] = m / (1.0 + jnp.exp(-p))
                regr_ref[pl.ds(d0, _DS)] = m / (1.0 + jnp.exp(-q))
            return 0

        jax.lax.fori_loop(0, nslab, slab_body, 0)


def kernel(data, w1, w2, w3, w4, w5):
    B, S = data.shape[0], data.shape[1]
    wflat = jnp.concatenate([
        w1.reshape(-1), w2.reshape(-1), w3.reshape(-1), w4.reshape(-1),
        w5.reshape(-1)])
    SG = S + _G
    SD = S + 2 * _PD
    out_shape = jax.ShapeDtypeStruct((S, S, S), jnp.float32)
    call = pl.pallas_call(
        functools.partial(_encoder_kernel, S),
        in_specs=[
            pl.BlockSpec((S, S, S), lambda: (0, 0, 0)),
            pl.BlockSpec(memory_space=pltpu.SMEM),
        ],
        out_specs=[
            pl.BlockSpec((S, S, S), lambda: (0, 0, 0)),
            pl.BlockSpec((S, S, S), lambda: (0, 0, 0)),
        ],
        out_shape=[out_shape, out_shape],
        scratch_shapes=[
            pltpu.VMEM((2, SD, SG, SG), jnp.float32),
            pltpu.VMEM((2, SD, SG, SG), jnp.float32),
        ],
    )
    probs = []
    regrs = []
    for b in range(B):
        p, q = call(data[b], wflat)
        probs.append(p)
        regrs.append(q)
    prob = jnp.stack(probs)[:, None]
    regr = jnp.stack(regrs)[:, None]
    return (prob, regr)
